# baseline (speedup 1.0000x reference)
import functools
import math

import numpy as np
import jax
import jax.numpy as jnp
from jax import lax
from jax.experimental import pallas as pl
from jax.experimental.pallas import tpu as pltpu
from jax.experimental.pallas import tpu_sc as plsc

N_NODES = 10000
N_EDGES = 160000
C = 16
LM = 9
N_DIST = 16
N_IRREPS = 3
_SLICES = [(0, 1), (1, 4), (4, 9)]
_PATHS = [(0, 0, 0), (0, 1, 1), (0, 2, 2), (1, 0, 1), (1, 1, 0), (1, 1, 2),
          (1, 2, 1), (2, 0, 2), (2, 1, 1), (2, 2, 0), (2, 2, 2)]
_IR_OF = [0, 1, 1, 1, 2, 2, 2, 2, 2]


def _su2_cg(j1, m1, j2, m2, j3, m3):
    if m3 != m1 + m2:
        return 0.0
    f = lambda n: math.factorial(int(round(n)))
    vmin = int(max(-j1 + j2 + m3, -j1 + m1, 0))
    vmax = int(min(j2 + j3 + m1, j3 - j1 + j2, j3 + m3))
    Cc = math.sqrt((2 * j3 + 1) * f(j3 + j1 - j2) * f(j3 - j1 + j2) * f(j1 + j2 - j3)
                   * f(j3 + m3) * f(j3 - m3)
                   / (f(j1 + j2 + j3 + 1) * f(j1 - m1) * f(j1 + m1) * f(j2 - m2) * f(j2 + m2)))
    S = 0.0
    for v in range(vmin, vmax + 1):
        S += (-1.0) ** (v + j2 + m2) * f(j2 + j3 + m1 - v) * f(j1 - m1 + v) / (
            f(v) * f(j3 - j1 + j2 - v) * f(j3 + m3 - v) * f(v + j1 - j2 - m3))
    return Cc * S


def _su2_cg_matrix(j1, j2, j3):
    mat = np.zeros((2 * j1 + 1, 2 * j2 + 1, 2 * j3 + 1))
    for m1 in range(-j1, j1 + 1):
        for m2 in range(-j2, j2 + 1):
            m3 = m1 + m2
            if abs(m3) <= j3:
                mat[j1 + m1, j2 + m2, j3 + m3] = _su2_cg(j1, m1, j2, m2, j3, m3)
    return mat / math.sqrt(2 * j3 + 1)


def _q_real_to_complex(l):
    q = np.zeros((2 * l + 1, 2 * l + 1), dtype=np.complex128)
    for m in range(-l, 0):
        q[l + m, l + abs(m)] = 1.0 / math.sqrt(2)
        q[l + m, l - abs(m)] = -1j / math.sqrt(2)
    q[l, l] = 1.0
    for m in range(1, l + 1):
        q[l + m, l + abs(m)] = (-1) ** m / math.sqrt(2)
        q[l + m, l - abs(m)] = 1j * (-1) ** m / math.sqrt(2)
    return (-1j) ** l * q


def _real_cg(l1, l2, l3):
    Cc = _su2_cg_matrix(l1, l2, l3).astype(np.complex128)
    Q1 = _q_real_to_complex(l1)
    Q2 = _q_real_to_complex(l2)
    Q3 = np.conj(_q_real_to_complex(l3).T)
    out = np.einsum('ij,kl,mn,ikn->jlm', Q1, Q2, Q3, Cc)
    return np.real(out).astype(np.float32)


_TP_NNZ = []
for _p, (_l1, _l2, _l3) in enumerate(_PATHS):
    _cg = _real_cg(_l1, _l2, _l3)
    _a0 = _SLICES[_l1][0]
    _b0 = _SLICES[_l2][0]
    _c0 = _SLICES[_l3][0]
    for _i in range(_cg.shape[0]):
        for _j in range(_cg.shape[1]):
            for _k in range(_cg.shape[2]):
                _v = float(_cg[_i, _j, _k])
                if _v != 0.0:
                    _TP_NNZ.append((_p, _a0 + _i, _b0 + _j, _c0 + _k, _v))
assert len({k for (_, _, _, k, _) in _TP_NNZ}) == LM


def _edge_math_T(nT, yT, distT, AB, lina2Wr, lina2Br, cmab_col,
                 WBcol, BBcol, linbWr, linbBr):
    B = nT.shape[1]
    z = jnp.concatenate([yT[i:i + 1] * yT for i in range(LM)]
                        + [yT, jnp.ones((1, B), jnp.float32)], axis=0)
    sAB = jnp.dot(AB, z, preferred_element_type=jnp.float32)
    sYaT = sAB[:LM]
    sYbT = sAB[LM:]
    waT2 = jnp.dot(lina2Wr, distT, preferred_element_type=jnp.float32) + lina2Br
    wbT = jnp.dot(linbWr, distT, preferred_element_type=jnp.float32) + linbBr
    n0 = nT[0:C]
    t_m = [waT2[m * C:(m + 1) * C] * n0 for m in range(N_IRREPS)]
    abias = (waT2[3 * C:4 * C] * n0) * cmab_col
    a_rows = []
    for l in range(LM):
        r = t_m[_IR_OF[l]] * sYaT[l:l + 1]
        if l == 0:
            r = r + abias
        a_rows.append(r)
    aT = jnp.concatenate(a_rows, axis=0)
    YbT_j = [WBcol[j * C:(j + 1) * C] * sYbT[j:j + 1] + BBcol[j * C:(j + 1) * C]
             for j in range(LM)]
    prods = {}
    for (p, i, j, k, v) in _TP_NNZ:
        if (i, j) not in prods:
            prods[(i, j)] = nT[i * C:(i + 1) * C] * YbT_j[j]
    slots = [None] * LM
    for (p, i, j, k, v) in _TP_NNZ:
        t = (v * wbT[p * C:(p + 1) * C]) * prods[(i, j)]
        slots[k] = t if slots[k] is None else slots[k] + t
    bT = jnp.concatenate(slots, axis=0)
    return aT + bT


_BT = 3200


def _k2_body(na_ref, nb_ref, y_ref, dist_ref,
             AB_ref, lina2Wr_ref, lina2Br_ref, cmab_ref,
             WBcol_ref, BBcol_ref, linbWr_ref, linbBr_ref,
             out_a_ref, out_b_ref):
    nT = jnp.concatenate([na_ref[...].T, nb_ref[...].T[:_WB]], axis=0)
    msgT = _edge_math_T(
        nT, y_ref[...].T, dist_ref[...].T,
        AB_ref[...], lina2Wr_ref[...], lina2Br_ref[...], cmab_ref[...],
        WBcol_ref[...], BBcol_ref[...], linbWr_ref[...], linbBr_ref[...])
    out_a_ref[...] = msgT[:_WA].T
    zb = jnp.zeros((_WA - _WB, _BT), jnp.float32)
    out_b_ref[...] = jnp.concatenate([msgT[_WA:], zb], axis=0).T


def _run_k2(nb_a, nb_b, y2, dist, small):
    nblk = N_EDGES // _BT
    em_spec = pl.BlockSpec((_BT, _WA), lambda b: (b, 0))
    col_spec = lambda cols: pl.BlockSpec((_BT, cols), lambda b: (b, 0))
    full = lambda a: pl.BlockSpec(a.shape, lambda b: tuple([0] * a.ndim))
    in_specs = [em_spec, em_spec, col_spec(LM), col_spec(N_DIST)] + [full(a) for a in small]
    return pl.pallas_call(
        _k2_body,
        grid=(nblk,),
        in_specs=in_specs,
        out_specs=(em_spec, em_spec),
        out_shape=(jax.ShapeDtypeStruct((N_EDGES, _WA), jnp.float32),
                   jax.ShapeDtypeStruct((N_EDGES, _WA), jnp.float32)),
        compiler_params=pltpu.CompilerParams(
            dimension_semantics=("parallel",)),
    )(nb_a, nb_b, y2, dist, *small)


_CH = 128
_NCHUNK = N_EDGES // _CH
_WA = 128
_WB = C * LM - _WA


_NW = 32
_K1_FULL = _NCHUNK // _NW
_K1_REM = _NCHUNK - _K1_FULL * _NW


def _k1_body(snd_a_hbm, snd_b_hbm, src_hbm, out_a_hbm, out_b_hbm,
             idx_v, rows_a, rows_b, sem_a0, sem_b0, sem_a1, sem_b1):
    ci = lax.axis_index("c")
    s = lax.axis_index("s")
    w = s * 2 + ci
    sems = [(sem_a0, sem_b0), (sem_a1, sem_b1)]

    def base_of(g):
        return (w + g * _NW) * _CH

    def fire(g, par):
        sa, sb = sems[par]
        base = base_of(g)
        pltpu.sync_copy(src_hbm.at[pl.ds(base, _CH)], idx_v.at[par])
        pltpu.async_copy(snd_a_hbm.at[idx_v.at[par]], rows_a.at[par], sa)
        pltpu.async_copy(snd_b_hbm.at[idx_v.at[par]], rows_b.at[par], sb)

    def drain_write(g, par):
        sa, sb = sems[par]
        pltpu.make_async_copy(snd_a_hbm.at[idx_v.at[par]], rows_a.at[par], sa).wait()
        pltpu.make_async_copy(snd_b_hbm.at[idx_v.at[par]], rows_b.at[par], sb).wait()
        base = base_of(g)
        pltpu.sync_copy(rows_a.at[par], out_a_hbm.at[pl.ds(base, _CH)])
        pltpu.sync_copy(rows_b.at[par], out_b_hbm.at[pl.ds(base, _CH)])

    fire(0, 0)

    def pair(i, _):
        g0 = 2 * i
        fire(g0 + 1, 1)
        drain_write(g0, 0)
        fire(g0 + 2, 0)
        drain_write(g0 + 1, 1)
        return _

    lax.fori_loop(0, (_K1_FULL - 1) // 2, pair, None)
    drain_write(_K1_FULL - 1, (_K1_FULL - 1) % 2)

    @pl.when(w < _K1_REM)
    def _():
        base = (_K1_FULL * _NW + w) * _CH
        pltpu.sync_copy(src_hbm.at[pl.ds(base, _CH)], idx_v.at[0])
        pltpu.async_copy(snd_a_hbm.at[idx_v.at[0]], rows_a.at[0], sem_a0).wait()
        pltpu.async_copy(snd_b_hbm.at[idx_v.at[0]], rows_b.at[0], sem_b0).wait()
        pltpu.sync_copy(rows_a.at[0], out_a_hbm.at[pl.ds(base, _CH)])
        pltpu.sync_copy(rows_b.at[0], out_b_hbm.at[pl.ds(base, _CH)])


def _run_k1(snd_a, snd_b, src):
    mesh = plsc.VectorSubcoreMesh(core_axis_name="c", subcore_axis_name="s")
    k = functools.partial(
        pl.kernel,
        out_type=(jax.ShapeDtypeStruct((N_EDGES, _WA), jnp.float32),
                  jax.ShapeDtypeStruct((N_EDGES, _WA), jnp.float32)),
        mesh=mesh,
        scratch_types=[
            pltpu.VMEM((2, _CH), jnp.int32),
            pltpu.VMEM((2, _CH, _WA), jnp.float32),
            pltpu.VMEM((2, _CH, _WA), jnp.float32),
            pltpu.SemaphoreType.DMA,
            pltpu.SemaphoreType.DMA,
            pltpu.SemaphoreType.DMA,
            pltpu.SemaphoreType.DMA,
        ],
    )(_k1_body)
    return k(snd_a, snd_b, src)


_NS = 16
_NPAD = 10240
_ROWS_PER_TILE = _NPAD // _NS
_K3_FULL = _NCHUNK // _NS
_K3_REM = _NCHUNK - _K3_FULL * _NS


def _k3_body(msg_a_hbm, msg_b_hbm, dst_hbm, rec_a_hbm, rec_b_hbm,
             out_hbm, acc, idx2, rows_v, sem0, sem1):
    ci = lax.axis_index("c")
    s = lax.axis_index("s")
    r0 = s * _ROWS_PER_TILE
    sems = [sem0, sem1]

    def scan_edges(msg_hbm, rec_hbm):
        pltpu.sync_copy(rec_hbm.at[pl.ds(r0, _ROWS_PER_TILE)],
                        acc.at[pl.ds(r0, _ROWS_PER_TILE)])
        plsc.subcore_barrier()

        def fire(g, par):
            base = (s + g * _NS) * _CH
            pltpu.sync_copy(dst_hbm.at[pl.ds(base, _CH)], idx2.at[par])
            pltpu.async_copy(msg_hbm.at[pl.ds(base, _CH)], rows_v.at[par], sems[par])

        def drain_scatter(g, par):
            base = (s + g * _NS) * _CH
            pltpu.make_async_copy(msg_hbm.at[pl.ds(base, _CH)],
                                  rows_v.at[par], sems[par]).wait()
            pltpu.sync_copy(rows_v.at[par], acc.at[idx2.at[par]], add=True)

        fire(0, 0)

        def pair(i, _):
            g0 = 2 * i
            fire(g0 + 1, 1)
            drain_scatter(g0, 0)

            @pl.when(g0 + 2 < _K3_FULL)
            def _():
                fire(g0 + 2, 0)

            drain_scatter(g0 + 1, 1)
            return _

        lax.fori_loop(0, _K3_FULL // 2, pair, None)

        @pl.when(s < _K3_REM)
        def _():
            base = (_K3_FULL * _NS + s) * _CH
            pltpu.sync_copy(dst_hbm.at[pl.ds(base, _CH)], idx2.at[0])
            pltpu.sync_copy(msg_hbm.at[pl.ds(base, _CH)], rows_v.at[0])
            pltpu.sync_copy(rows_v.at[0], acc.at[idx2.at[0]], add=True)

        plsc.subcore_barrier()
        pltpu.sync_copy(acc.at[pl.ds(r0, _ROWS_PER_TILE)],
                        out_hbm.at[ci, pl.ds(r0, _ROWS_PER_TILE)])

    @pl.when(ci == 0)
    def _():
        scan_edges(msg_a_hbm, rec_a_hbm)

    @pl.when(ci == 1)
    def _():
        scan_edges(msg_b_hbm, rec_b_hbm)


def _run_k3(msg_a, msg_b, dst, rec_a, rec_b):
    mesh = plsc.VectorSubcoreMesh(core_axis_name="c", subcore_axis_name="s")
    k = functools.partial(
        pl.kernel,
        out_type=jax.ShapeDtypeStruct((2, _NPAD, _WA), jnp.float32),
        mesh=mesh,
        scratch_types=[
            pltpu.VMEM_SHARED((_NPAD, _WA), jnp.float32),
            pltpu.VMEM((2, _CH), jnp.int32),
            pltpu.VMEM((2, _CH, _WA), jnp.float32),
            pltpu.SemaphoreType.DMA,
            pltpu.SemaphoreType.DMA,
        ],
    )(_k3_body)
    return k(msg_a, msg_b, dst, rec_a, rec_b)


_NNZ_P = np.array([p for (p, i, j, k, v) in _TP_NNZ])
_NNZ_K = np.array([k for (p, i, j, k, v) in _TP_NNZ])
_NNZ_C = np.array([i * LM + j for (p, i, j, k, v) in _TP_NNZ])
_NNZ_V = np.array([v for (p, i, j, k, v) in _TP_NNZ], np.float32)


def _selfmix_matrix(pw, b0, kk):
    A = jnp.zeros((LM, 91), jnp.float32)
    A = A.at[_NNZ_K, _NNZ_C].add(_NNZ_V * pw[_NNZ_P])
    A = A.at[np.arange(LM), 81 + np.arange(LM)].add(kk[np.array(_IR_OF)])
    A = A.at[0, 90].add(b0[0])
    return A


def _prep_small(sma_path_w, sma_bias0, sma_k, cma_W, cma_b, lina_W, lina_b,
                smb_path_w, smb_bias0, smb_k, cmb_W, cmb_b, linb_W, linb_b):
    ir = jnp.asarray(_IR_OF)
    NP = len(_PATHS)
    AB = jnp.concatenate([_selfmix_matrix(sma_path_w, sma_bias0, sma_k[0]),
                          _selfmix_matrix(smb_path_w, smb_bias0, smb_k[0])], axis=0)
    WBcol = cmb_W[ir, 0, :].reshape(C * LM, 1)
    BBcol = jnp.concatenate([cmb_b, jnp.zeros(C * (LM - 1), jnp.float32)]).reshape(C * LM, 1)
    linaWr = lina_W.reshape(N_DIST, C, N_IRREPS).transpose(2, 1, 0).reshape(N_IRREPS * C, N_DIST)
    linaBr = lina_b.reshape(C, N_IRREPS).T.reshape(N_IRREPS * C, 1)
    scaleA = cma_W[:, 0, :].reshape(N_IRREPS * C, 1)
    lina2Wr = jnp.concatenate([linaWr * scaleA, linaWr[0:C]], axis=0)
    lina2Br = jnp.concatenate([linaBr * scaleA, linaBr[0:C]], axis=0)
    cmab_col = cma_b.reshape(C, 1)
    linbWr = linb_W.reshape(N_DIST, C, NP).transpose(2, 1, 0).reshape(NP * C, N_DIST)
    linbBr = linb_b.reshape(C, NP).T.reshape(NP * C, 1)
    return [AB, lina2Wr, lina2Br, cmab_col, WBcol, BBcol, linbWr, linbBr]


def kernel(nodes_rec, nodes_snd, edge_ind, Y_edge, dist_feat,
           sma_path_w, sma_bias0, sma_k, cma_W, cma_b, lina_W, lina_b,
           smb_path_w, smb_bias0, smb_k, cmb_W, cmb_b, linb_W, linb_b):
    src = edge_ind[:, 0].astype(jnp.int32)
    dst = edge_ind[:, 1].astype(jnp.int32)
    snd_flat = nodes_snd.transpose(0, 2, 1).reshape(N_NODES, C * LM)
    rec_flat = nodes_rec.transpose(0, 2, 1).reshape(N_NODES, C * LM)
    snd_a = snd_flat[:, :_WA]
    snd_b = jnp.pad(snd_flat[:, _WA:], ((0, 0), (0, _WA - _WB)))

    nb_a, nb_b = _run_k1(snd_a, snd_b, src)
    y2 = Y_edge.reshape(N_EDGES, LM)
    small = _prep_small(sma_path_w, sma_bias0, sma_k, cma_W, cma_b, lina_W, lina_b,
                        smb_path_w, smb_bias0, smb_k, cmb_W, cmb_b, linb_W, linb_b)
    msg_a, msg_b = _run_k2(nb_a, nb_b, y2, dist_feat, small)
    rec_a = jnp.pad(rec_flat[:, :_WA], ((0, _NPAD - N_NODES), (0, 0)))
    rec_b = jnp.pad(rec_flat[:, _WA:], ((0, _NPAD - N_NODES), (0, _WA - _WB)))
    partials = _run_k3(msg_a, msg_b, dst, rec_a, rec_b)
    out_flat = jnp.concatenate(
        [partials[0, :N_NODES], partials[1, :N_NODES, :_WB]], axis=1)
    return out_flat.reshape(N_NODES, LM, C).transpose(0, 2, 1)


# speedup vs baseline: 1.1832x; 1.1832x over previous
import functools
import math

import numpy as np
import jax
import jax.numpy as jnp
from jax import lax
from jax.experimental import pallas as pl
from jax.experimental.pallas import tpu as pltpu
from jax.experimental.pallas import tpu_sc as plsc

N_NODES = 10000
N_EDGES = 160000
C = 16
LM = 9
N_DIST = 16
N_IRREPS = 3
_SLICES = [(0, 1), (1, 4), (4, 9)]
_PATHS = [(0, 0, 0), (0, 1, 1), (0, 2, 2), (1, 0, 1), (1, 1, 0), (1, 1, 2),
          (1, 2, 1), (2, 0, 2), (2, 1, 1), (2, 2, 0), (2, 2, 2)]
_IR_OF = [0, 1, 1, 1, 2, 2, 2, 2, 2]


def _su2_cg(j1, m1, j2, m2, j3, m3):
    if m3 != m1 + m2:
        return 0.0
    f = lambda n: math.factorial(int(round(n)))
    vmin = int(max(-j1 + j2 + m3, -j1 + m1, 0))
    vmax = int(min(j2 + j3 + m1, j3 - j1 + j2, j3 + m3))
    Cc = math.sqrt((2 * j3 + 1) * f(j3 + j1 - j2) * f(j3 - j1 + j2) * f(j1 + j2 - j3)
                   * f(j3 + m3) * f(j3 - m3)
                   / (f(j1 + j2 + j3 + 1) * f(j1 - m1) * f(j1 + m1) * f(j2 - m2) * f(j2 + m2)))
    S = 0.0
    for v in range(vmin, vmax + 1):
        S += (-1.0) ** (v + j2 + m2) * f(j2 + j3 + m1 - v) * f(j1 - m1 + v) / (
            f(v) * f(j3 - j1 + j2 - v) * f(j3 + m3 - v) * f(v + j1 - j2 - m3))
    return Cc * S


def _su2_cg_matrix(j1, j2, j3):
    mat = np.zeros((2 * j1 + 1, 2 * j2 + 1, 2 * j3 + 1))
    for m1 in range(-j1, j1 + 1):
        for m2 in range(-j2, j2 + 1):
            m3 = m1 + m2
            if abs(m3) <= j3:
                mat[j1 + m1, j2 + m2, j3 + m3] = _su2_cg(j1, m1, j2, m2, j3, m3)
    return mat / math.sqrt(2 * j3 + 1)


def _q_real_to_complex(l):
    q = np.zeros((2 * l + 1, 2 * l + 1), dtype=np.complex128)
    for m in range(-l, 0):
        q[l + m, l + abs(m)] = 1.0 / math.sqrt(2)
        q[l + m, l - abs(m)] = -1j / math.sqrt(2)
    q[l, l] = 1.0
    for m in range(1, l + 1):
        q[l + m, l + abs(m)] = (-1) ** m / math.sqrt(2)
        q[l + m, l - abs(m)] = 1j * (-1) ** m / math.sqrt(2)
    return (-1j) ** l * q


def _real_cg(l1, l2, l3):
    Cc = _su2_cg_matrix(l1, l2, l3).astype(np.complex128)
    Q1 = _q_real_to_complex(l1)
    Q2 = _q_real_to_complex(l2)
    Q3 = np.conj(_q_real_to_complex(l3).T)
    out = np.einsum('ij,kl,mn,ikn->jlm', Q1, Q2, Q3, Cc)
    return np.real(out).astype(np.float32)


_TP_NNZ = []
for _p, (_l1, _l2, _l3) in enumerate(_PATHS):
    _cg = _real_cg(_l1, _l2, _l3)
    _a0 = _SLICES[_l1][0]
    _b0 = _SLICES[_l2][0]
    _c0 = _SLICES[_l3][0]
    for _i in range(_cg.shape[0]):
        for _j in range(_cg.shape[1]):
            for _k in range(_cg.shape[2]):
                _v = float(_cg[_i, _j, _k])
                if _v != 0.0:
                    _TP_NNZ.append((_p, _a0 + _i, _b0 + _j, _c0 + _k, _v))
assert len({k for (_, _, _, k, _) in _TP_NNZ}) == LM


def _edge_math_T(nT, yT, distT, AB, lina2Wr, lina2Br, cmab_col,
                 WBcol, BBcol, linbWr, linbBr):
    B = nT.shape[1]
    z = jnp.concatenate([yT[i:i + 1] * yT for i in range(LM)]
                        + [yT, jnp.ones((1, B), jnp.float32)], axis=0)
    sAB = jnp.dot(AB, z, preferred_element_type=jnp.float32)
    sYaT = sAB[:LM]
    sYbT = sAB[LM:]
    waT2 = jnp.dot(lina2Wr, distT, preferred_element_type=jnp.float32) + lina2Br
    wbT = jnp.dot(linbWr, distT, preferred_element_type=jnp.float32) + linbBr
    n0 = nT[0:C]
    t_m = [waT2[m * C:(m + 1) * C] * n0 for m in range(N_IRREPS)]
    abias = (waT2[3 * C:4 * C] * n0) * cmab_col
    a_rows = []
    for l in range(LM):
        r = t_m[_IR_OF[l]] * sYaT[l:l + 1]
        if l == 0:
            r = r + abias
        a_rows.append(r)
    aT = jnp.concatenate(a_rows, axis=0)
    YbT_j = [WBcol[j * C:(j + 1) * C] * sYbT[j:j + 1] + BBcol[j * C:(j + 1) * C]
             for j in range(LM)]
    prods = {}
    for (p, i, j, k, v) in _TP_NNZ:
        if (i, j) not in prods:
            prods[(i, j)] = nT[i * C:(i + 1) * C] * YbT_j[j]
    slots = [None] * LM
    for (p, i, j, k, v) in _TP_NNZ:
        t = (v * wbT[p * C:(p + 1) * C]) * prods[(i, j)]
        slots[k] = t if slots[k] is None else slots[k] + t
    bT = jnp.concatenate(slots, axis=0)
    return aT + bT


_BT = 3200


def _k2_body(na_ref, nb_ref, yT_ref, distT_ref,
             AB_ref, lina2Wr_ref, lina2Br_ref, cmab_ref,
             WBcol_ref, BBcol_ref, linbWr_ref, linbBr_ref,
             out_a_ref, out_b_ref):
    nT = jnp.concatenate([na_ref[...].T, nb_ref[...].T[:_WB]], axis=0)
    msgT = _edge_math_T(
        nT, yT_ref[...], distT_ref[...],
        AB_ref[...], lina2Wr_ref[...], lina2Br_ref[...], cmab_ref[...],
        WBcol_ref[...], BBcol_ref[...], linbWr_ref[...], linbBr_ref[...])
    out_a_ref[...] = msgT[:_WA].T
    out_b_ref[:, 0:_WB] = msgT[_WA:].T
    out_b_ref[:, _WB:_WA] = jnp.zeros((_BT, _WA - _WB), jnp.float32)


def _run_k2(nb_a, nb_b, yT, distT, small):
    nblk = N_EDGES // _BT
    em_spec = pl.BlockSpec((_BT, _WA), lambda b: (b, 0))
    cm_spec = lambda rows: pl.BlockSpec((rows, _BT), lambda b: (0, b))
    full = lambda a: pl.BlockSpec(a.shape, lambda b: tuple([0] * a.ndim))
    in_specs = [em_spec, em_spec, cm_spec(LM), cm_spec(N_DIST)] + [full(a) for a in small]
    return pl.pallas_call(
        _k2_body,
        grid=(nblk,),
        in_specs=in_specs,
        out_specs=(em_spec, em_spec),
        out_shape=(jax.ShapeDtypeStruct((N_EDGES, _WA), jnp.float32),
                   jax.ShapeDtypeStruct((N_EDGES, _WA), jnp.float32)),
        compiler_params=pltpu.CompilerParams(
            dimension_semantics=("parallel",)),
    )(nb_a, nb_b, yT, distT, *small)


_CH = 128
_NCHUNK = N_EDGES // _CH
_WA = 128
_WB = C * LM - _WA


_NW = 32
_K1_FULL = _NCHUNK // _NW
_K1_REM = _NCHUNK - _K1_FULL * _NW


def _k1_body(snd_a_hbm, snd_b_hbm, src_hbm, out_a_hbm, out_b_hbm,
             idx_v, rows_a, rows_b, sem_a0, sem_b0, sem_a1, sem_b1):
    ci = lax.axis_index("c")
    s = lax.axis_index("s")
    w = s * 2 + ci
    sems = [(sem_a0, sem_b0), (sem_a1, sem_b1)]

    def base_of(g):
        return (w + g * _NW) * _CH

    def fire(g, par):
        sa, sb = sems[par]
        base = base_of(g)
        pltpu.sync_copy(src_hbm.at[pl.ds(base, _CH)], idx_v.at[par])
        pltpu.async_copy(snd_a_hbm.at[idx_v.at[par]], rows_a.at[par], sa)
        pltpu.async_copy(snd_b_hbm.at[idx_v.at[par]], rows_b.at[par], sb)

    def drain_write(g, par):
        sa, sb = sems[par]
        pltpu.make_async_copy(snd_a_hbm.at[idx_v.at[par]], rows_a.at[par], sa).wait()
        pltpu.make_async_copy(snd_b_hbm.at[idx_v.at[par]], rows_b.at[par], sb).wait()
        base = base_of(g)
        pltpu.sync_copy(rows_a.at[par], out_a_hbm.at[pl.ds(base, _CH)])
        pltpu.sync_copy(rows_b.at[par], out_b_hbm.at[pl.ds(base, _CH)])

    fire(0, 0)

    def pair(i, _):
        g0 = 2 * i
        fire(g0 + 1, 1)
        drain_write(g0, 0)
        fire(g0 + 2, 0)
        drain_write(g0 + 1, 1)
        return _

    lax.fori_loop(0, (_K1_FULL - 1) // 2, pair, None)
    drain_write(_K1_FULL - 1, (_K1_FULL - 1) % 2)

    @pl.when(w < _K1_REM)
    def _():
        base = (_K1_FULL * _NW + w) * _CH
        pltpu.sync_copy(src_hbm.at[pl.ds(base, _CH)], idx_v.at[0])
        pltpu.async_copy(snd_a_hbm.at[idx_v.at[0]], rows_a.at[0], sem_a0).wait()
        pltpu.async_copy(snd_b_hbm.at[idx_v.at[0]], rows_b.at[0], sem_b0).wait()
        pltpu.sync_copy(rows_a.at[0], out_a_hbm.at[pl.ds(base, _CH)])
        pltpu.sync_copy(rows_b.at[0], out_b_hbm.at[pl.ds(base, _CH)])


def _run_k1(snd_a, snd_b, src):
    mesh = plsc.VectorSubcoreMesh(core_axis_name="c", subcore_axis_name="s")
    k = functools.partial(
        pl.kernel,
        out_type=(jax.ShapeDtypeStruct((N_EDGES, _WA), jnp.float32),
                  jax.ShapeDtypeStruct((N_EDGES, _WA), jnp.float32)),
        mesh=mesh,
        scratch_types=[
            pltpu.VMEM((2, _CH), jnp.int32),
            pltpu.VMEM((2, _CH, _WA), jnp.float32),
            pltpu.VMEM((2, _CH, _WA), jnp.float32),
            pltpu.SemaphoreType.DMA,
            pltpu.SemaphoreType.DMA,
            pltpu.SemaphoreType.DMA,
            pltpu.SemaphoreType.DMA,
        ],
    )(_k1_body)
    return k(snd_a, snd_b, src)


_NS = 16
_NPAD = 10240
_ROWS_PER_TILE = _NPAD // _NS
_K3_FULL = _NCHUNK // _NS
_K3_REM = _NCHUNK - _K3_FULL * _NS


def _k3_body(msg_a_hbm, msg_b_hbm, dst_hbm, rec_a_hbm, rec_b_hbm,
             out_hbm, acc, idx2, rows_v, sem0, sem1):
    ci = lax.axis_index("c")
    s = lax.axis_index("s")
    r0 = s * _ROWS_PER_TILE
    sems = [sem0, sem1]

    def scan_edges(msg_hbm, rec_hbm):
        pltpu.sync_copy(rec_hbm.at[pl.ds(r0, _ROWS_PER_TILE)],
                        acc.at[pl.ds(r0, _ROWS_PER_TILE)])
        plsc.subcore_barrier()

        def fire(g, par):
            base = (s + g * _NS) * _CH
            pltpu.sync_copy(dst_hbm.at[pl.ds(base, _CH)], idx2.at[par])
            pltpu.async_copy(msg_hbm.at[pl.ds(base, _CH)], rows_v.at[par], sems[par])

        def drain_scatter(g, par):
            base = (s + g * _NS) * _CH
            pltpu.make_async_copy(msg_hbm.at[pl.ds(base, _CH)],
                                  rows_v.at[par], sems[par]).wait()
            pltpu.sync_copy(rows_v.at[par], acc.at[idx2.at[par]], add=True)

        fire(0, 0)

        def pair(i, _):
            g0 = 2 * i
            fire(g0 + 1, 1)
            drain_scatter(g0, 0)

            @pl.when(g0 + 2 < _K3_FULL)
            def _():
                fire(g0 + 2, 0)

            drain_scatter(g0 + 1, 1)
            return _

        lax.fori_loop(0, _K3_FULL // 2, pair, None)

        @pl.when(s < _K3_REM)
        def _():
            base = (_K3_FULL * _NS + s) * _CH
            pltpu.sync_copy(dst_hbm.at[pl.ds(base, _CH)], idx2.at[0])
            pltpu.sync_copy(msg_hbm.at[pl.ds(base, _CH)], rows_v.at[0])
            pltpu.sync_copy(rows_v.at[0], acc.at[idx2.at[0]], add=True)

        plsc.subcore_barrier()
        pltpu.sync_copy(acc.at[pl.ds(r0, _ROWS_PER_TILE)],
                        out_hbm.at[ci, pl.ds(r0, _ROWS_PER_TILE)])

    @pl.when(ci == 0)
    def _():
        scan_edges(msg_a_hbm, rec_a_hbm)

    @pl.when(ci == 1)
    def _():
        scan_edges(msg_b_hbm, rec_b_hbm)


def _run_k3(msg_a, msg_b, dst, rec_a, rec_b):
    mesh = plsc.VectorSubcoreMesh(core_axis_name="c", subcore_axis_name="s")
    k = functools.partial(
        pl.kernel,
        out_type=jax.ShapeDtypeStruct((2, _NPAD, _WA), jnp.float32),
        mesh=mesh,
        scratch_types=[
            pltpu.VMEM_SHARED((_NPAD, _WA), jnp.float32),
            pltpu.VMEM((2, _CH), jnp.int32),
            pltpu.VMEM((2, _CH, _WA), jnp.float32),
            pltpu.SemaphoreType.DMA,
            pltpu.SemaphoreType.DMA,
        ],
    )(_k3_body)
    return k(msg_a, msg_b, dst, rec_a, rec_b)


_NNZ_P = np.array([p for (p, i, j, k, v) in _TP_NNZ])
_NNZ_K = np.array([k for (p, i, j, k, v) in _TP_NNZ])
_NNZ_C = np.array([i * LM + j for (p, i, j, k, v) in _TP_NNZ])
_NNZ_V = np.array([v for (p, i, j, k, v) in _TP_NNZ], np.float32)


def _selfmix_matrix(pw, b0, kk):
    A = jnp.zeros((LM, 91), jnp.float32)
    A = A.at[_NNZ_K, _NNZ_C].add(_NNZ_V * pw[_NNZ_P])
    A = A.at[np.arange(LM), 81 + np.arange(LM)].add(kk[np.array(_IR_OF)])
    A = A.at[0, 90].add(b0[0])
    return A


def _prep_small(sma_path_w, sma_bias0, sma_k, cma_W, cma_b, lina_W, lina_b,
                smb_path_w, smb_bias0, smb_k, cmb_W, cmb_b, linb_W, linb_b):
    ir = jnp.asarray(_IR_OF)
    NP = len(_PATHS)
    AB = jnp.concatenate([_selfmix_matrix(sma_path_w, sma_bias0, sma_k[0]),
                          _selfmix_matrix(smb_path_w, smb_bias0, smb_k[0])], axis=0)
    WBcol = cmb_W[ir, 0, :].reshape(C * LM, 1)
    BBcol = jnp.concatenate([cmb_b, jnp.zeros(C * (LM - 1), jnp.float32)]).reshape(C * LM, 1)
    linaWr = lina_W.reshape(N_DIST, C, N_IRREPS).transpose(2, 1, 0).reshape(N_IRREPS * C, N_DIST)
    linaBr = lina_b.reshape(C, N_IRREPS).T.reshape(N_IRREPS * C, 1)
    scaleA = cma_W[:, 0, :].reshape(N_IRREPS * C, 1)
    lina2Wr = jnp.concatenate([linaWr * scaleA, linaWr[0:C]], axis=0)
    lina2Br = jnp.concatenate([linaBr * scaleA, linaBr[0:C]], axis=0)
    cmab_col = cma_b.reshape(C, 1)
    linbWr = linb_W.reshape(N_DIST, C, NP).transpose(2, 1, 0).reshape(NP * C, N_DIST)
    linbBr = linb_b.reshape(C, NP).T.reshape(NP * C, 1)
    return [AB, lina2Wr, lina2Br, cmab_col, WBcol, BBcol, linbWr, linbBr]


def kernel(nodes_rec, nodes_snd, edge_ind, Y_edge, dist_feat,
           sma_path_w, sma_bias0, sma_k, cma_W, cma_b, lina_W, lina_b,
           smb_path_w, smb_bias0, smb_k, cmb_W, cmb_b, linb_W, linb_b):
    src = edge_ind[:, 0].astype(jnp.int32)
    dst = edge_ind[:, 1].astype(jnp.int32)
    snd_flat = nodes_snd.transpose(0, 2, 1).reshape(N_NODES, C * LM)
    rec_flat = nodes_rec.transpose(0, 2, 1).reshape(N_NODES, C * LM)
    snd_a = snd_flat[:, :_WA]
    snd_b = jnp.pad(snd_flat[:, _WA:], ((0, 0), (0, _WA - _WB)))

    nb_a, nb_b = _run_k1(snd_a, snd_b, src)
    yT = Y_edge.reshape(N_EDGES, LM).T
    distT = dist_feat.T
    small = _prep_small(sma_path_w, sma_bias0, sma_k, cma_W, cma_b, lina_W, lina_b,
                        smb_path_w, smb_bias0, smb_k, cmb_W, cmb_b, linb_W, linb_b)
    msg_a, msg_b = _run_k2(nb_a, nb_b, yT, distT, small)
    rec_a = jnp.pad(rec_flat[:, :_WA], ((0, _NPAD - N_NODES), (0, 0)))
    rec_b = jnp.pad(rec_flat[:, _WA:], ((0, _NPAD - N_NODES), (0, _WA - _WB)))
    partials = _run_k3(msg_a, msg_b, dst, rec_a, rec_b)
    out_flat = jnp.concatenate(
        [partials[0, :N_NODES], partials[1, :N_NODES, :_WB]], axis=1)
    return out_flat.reshape(N_NODES, LM, C).transpose(0, 2, 1)


# speedup vs baseline: 1.1986x; 1.0130x over previous
import functools
import math

import numpy as np
import jax
import jax.numpy as jnp
from jax import lax
from jax.experimental import pallas as pl
from jax.experimental.pallas import tpu as pltpu
from jax.experimental.pallas import tpu_sc as plsc

N_NODES = 10000
N_EDGES = 160000
C = 16
LM = 9
N_DIST = 16
N_IRREPS = 3
_SLICES = [(0, 1), (1, 4), (4, 9)]
_PATHS = [(0, 0, 0), (0, 1, 1), (0, 2, 2), (1, 0, 1), (1, 1, 0), (1, 1, 2),
          (1, 2, 1), (2, 0, 2), (2, 1, 1), (2, 2, 0), (2, 2, 2)]
_IR_OF = [0, 1, 1, 1, 2, 2, 2, 2, 2]


def _su2_cg(j1, m1, j2, m2, j3, m3):
    if m3 != m1 + m2:
        return 0.0
    f = lambda n: math.factorial(int(round(n)))
    vmin = int(max(-j1 + j2 + m3, -j1 + m1, 0))
    vmax = int(min(j2 + j3 + m1, j3 - j1 + j2, j3 + m3))
    Cc = math.sqrt((2 * j3 + 1) * f(j3 + j1 - j2) * f(j3 - j1 + j2) * f(j1 + j2 - j3)
                   * f(j3 + m3) * f(j3 - m3)
                   / (f(j1 + j2 + j3 + 1) * f(j1 - m1) * f(j1 + m1) * f(j2 - m2) * f(j2 + m2)))
    S = 0.0
    for v in range(vmin, vmax + 1):
        S += (-1.0) ** (v + j2 + m2) * f(j2 + j3 + m1 - v) * f(j1 - m1 + v) / (
            f(v) * f(j3 - j1 + j2 - v) * f(j3 + m3 - v) * f(v + j1 - j2 - m3))
    return Cc * S


def _su2_cg_matrix(j1, j2, j3):
    mat = np.zeros((2 * j1 + 1, 2 * j2 + 1, 2 * j3 + 1))
    for m1 in range(-j1, j1 + 1):
        for m2 in range(-j2, j2 + 1):
            m3 = m1 + m2
            if abs(m3) <= j3:
                mat[j1 + m1, j2 + m2, j3 + m3] = _su2_cg(j1, m1, j2, m2, j3, m3)
    return mat / math.sqrt(2 * j3 + 1)


def _q_real_to_complex(l):
    q = np.zeros((2 * l + 1, 2 * l + 1), dtype=np.complex128)
    for m in range(-l, 0):
        q[l + m, l + abs(m)] = 1.0 / math.sqrt(2)
        q[l + m, l - abs(m)] = -1j / math.sqrt(2)
    q[l, l] = 1.0
    for m in range(1, l + 1):
        q[l + m, l + abs(m)] = (-1) ** m / math.sqrt(2)
        q[l + m, l - abs(m)] = 1j * (-1) ** m / math.sqrt(2)
    return (-1j) ** l * q


def _real_cg(l1, l2, l3):
    Cc = _su2_cg_matrix(l1, l2, l3).astype(np.complex128)
    Q1 = _q_real_to_complex(l1)
    Q2 = _q_real_to_complex(l2)
    Q3 = np.conj(_q_real_to_complex(l3).T)
    out = np.einsum('ij,kl,mn,ikn->jlm', Q1, Q2, Q3, Cc)
    return np.real(out).astype(np.float32)


_TP_NNZ = []
for _p, (_l1, _l2, _l3) in enumerate(_PATHS):
    _cg = _real_cg(_l1, _l2, _l3)
    _a0 = _SLICES[_l1][0]
    _b0 = _SLICES[_l2][0]
    _c0 = _SLICES[_l3][0]
    for _i in range(_cg.shape[0]):
        for _j in range(_cg.shape[1]):
            for _k in range(_cg.shape[2]):
                _v = float(_cg[_i, _j, _k])
                if _v != 0.0:
                    _TP_NNZ.append((_p, _a0 + _i, _b0 + _j, _c0 + _k, _v))
assert len({k for (_, _, _, k, _) in _TP_NNZ}) == LM


def _edge_math_T(nT, yT, distT, AB, lina2Wr, lina2Br, cmab_col,
                 WBcol, BBcol, linbWr, linbBr):
    B = nT.shape[1]
    z = jnp.concatenate([yT[i:i + 1] * yT for i in range(LM)]
                        + [yT, jnp.ones((1, B), jnp.float32)], axis=0)
    sAB = jnp.dot(AB, z, preferred_element_type=jnp.float32)
    sYaT = sAB[:LM]
    sYbT = sAB[LM:]
    waT2 = jnp.dot(lina2Wr, distT, preferred_element_type=jnp.float32) + lina2Br
    wbT = jnp.dot(linbWr, distT, preferred_element_type=jnp.float32) + linbBr
    n0 = nT[0:C]
    t_m = [waT2[m * C:(m + 1) * C] * n0 for m in range(N_IRREPS)]
    abias = (waT2[3 * C:4 * C] * n0) * cmab_col
    a_rows = []
    for l in range(LM):
        r = t_m[_IR_OF[l]] * sYaT[l:l + 1]
        if l == 0:
            r = r + abias
        a_rows.append(r)
    aT = jnp.concatenate(a_rows, axis=0)
    YbT_j = [WBcol[j * C:(j + 1) * C] * sYbT[j:j + 1] + BBcol[j * C:(j + 1) * C]
             for j in range(LM)]
    prods = {}
    for (p, i, j, k, v) in _TP_NNZ:
        if (i, j) not in prods:
            prods[(i, j)] = nT[i * C:(i + 1) * C] * YbT_j[j]
    slots = [None] * LM
    for (p, i, j, k, v) in _TP_NNZ:
        t = (v * wbT[p * C:(p + 1) * C]) * prods[(i, j)]
        slots[k] = t if slots[k] is None else slots[k] + t
    bT = jnp.concatenate(slots, axis=0)
    return aT + bT


_BT = 3200


def _k2_body(na_ref, nb_ref, yT_ref, distT_ref,
             AB_ref, lina2Wr_ref, lina2Br_ref, cmab_ref,
             WBcol_ref, BBcol_ref, linbWr_ref, linbBr_ref,
             out_a_ref, out_b_ref):
    nT = jnp.concatenate([na_ref[...].T, nb_ref[...].T[:_WB]], axis=0)
    msgT = _edge_math_T(
        nT, yT_ref[...], distT_ref[...],
        AB_ref[...], lina2Wr_ref[...], lina2Br_ref[...], cmab_ref[...],
        WBcol_ref[...], BBcol_ref[...], linbWr_ref[...], linbBr_ref[...])
    out_a_ref[...] = msgT[:_WA].T
    out_b_ref[:, 0:_WB] = msgT[_WA:].T
    out_b_ref[:, _WB:_WA] = jnp.zeros((_BT, _WA - _WB), jnp.float32)


def _run_k2(nb_a, nb_b, yT, distT, small):
    es = nb_a.shape[0]
    nblk = es // _BT
    em_spec = pl.BlockSpec((_BT, _WA), lambda b: (b, 0))
    cm_spec = lambda rows: pl.BlockSpec((rows, _BT), lambda b: (0, b))
    full = lambda a: pl.BlockSpec(a.shape, lambda b: tuple([0] * a.ndim))
    in_specs = [em_spec, em_spec, cm_spec(LM), cm_spec(N_DIST)] + [full(a) for a in small]
    return pl.pallas_call(
        _k2_body,
        grid=(nblk,),
        in_specs=in_specs,
        out_specs=(em_spec, em_spec),
        out_shape=(jax.ShapeDtypeStruct((es, _WA), jnp.float32),
                   jax.ShapeDtypeStruct((es, _WA), jnp.float32)),
        compiler_params=pltpu.CompilerParams(
            dimension_semantics=("parallel",)),
    )(nb_a, nb_b, yT, distT, *small)


_CH = 128
_NCHUNK = N_EDGES // _CH
_WA = 128
_WB = C * LM - _WA


_NW = 32


def _run_k1(snd_a, snd_b, src):
    es = src.shape[0]
    nch = es // _CH
    full_r = nch // _NW
    rem = nch - full_r * _NW
    assert full_r % 2 == 1

    def body(snd_a_hbm, snd_b_hbm, src_hbm, out_a_hbm, out_b_hbm,
             idx_v, rows_a, rows_b, sem_a0, sem_b0, sem_a1, sem_b1):
        ci = lax.axis_index("c")
        s = lax.axis_index("s")
        w = s * 2 + ci
        sems = [(sem_a0, sem_b0), (sem_a1, sem_b1)]

        def base_of(g):
            return (w + g * _NW) * _CH

        def fire(g, par):
            sa, sb = sems[par]
            base = base_of(g)
            pltpu.sync_copy(src_hbm.at[pl.ds(base, _CH)], idx_v.at[par])
            pltpu.async_copy(snd_a_hbm.at[idx_v.at[par]], rows_a.at[par], sa)
            pltpu.async_copy(snd_b_hbm.at[idx_v.at[par]], rows_b.at[par], sb)

        def drain_write(g, par):
            sa, sb = sems[par]
            pltpu.make_async_copy(snd_a_hbm.at[idx_v.at[par]], rows_a.at[par], sa).wait()
            pltpu.make_async_copy(snd_b_hbm.at[idx_v.at[par]], rows_b.at[par], sb).wait()
            base = base_of(g)
            pltpu.sync_copy(rows_a.at[par], out_a_hbm.at[pl.ds(base, _CH)])
            pltpu.sync_copy(rows_b.at[par], out_b_hbm.at[pl.ds(base, _CH)])

        fire(0, 0)

        def pair(i, _):
            g0 = 2 * i
            fire(g0 + 1, 1)
            drain_write(g0, 0)
            fire(g0 + 2, 0)
            drain_write(g0 + 1, 1)
            return _

        lax.fori_loop(0, (full_r - 1) // 2, pair, None)
        drain_write(full_r - 1, 0)

        @pl.when(w < rem)
        def _():
            base = (full_r * _NW + w) * _CH
            pltpu.sync_copy(src_hbm.at[pl.ds(base, _CH)], idx_v.at[0])
            pltpu.async_copy(snd_a_hbm.at[idx_v.at[0]], rows_a.at[0], sem_a0).wait()
            pltpu.async_copy(snd_b_hbm.at[idx_v.at[0]], rows_b.at[0], sem_b0).wait()
            pltpu.sync_copy(rows_a.at[0], out_a_hbm.at[pl.ds(base, _CH)])
            pltpu.sync_copy(rows_b.at[0], out_b_hbm.at[pl.ds(base, _CH)])

    mesh = plsc.VectorSubcoreMesh(core_axis_name="c", subcore_axis_name="s")
    k = functools.partial(
        pl.kernel,
        out_type=(jax.ShapeDtypeStruct((es, _WA), jnp.float32),
                  jax.ShapeDtypeStruct((es, _WA), jnp.float32)),
        mesh=mesh,
        scratch_types=[
            pltpu.VMEM((2, _CH), jnp.int32),
            pltpu.VMEM((2, _CH, _WA), jnp.float32),
            pltpu.VMEM((2, _CH, _WA), jnp.float32),
            pltpu.SemaphoreType.DMA,
            pltpu.SemaphoreType.DMA,
            pltpu.SemaphoreType.DMA,
            pltpu.SemaphoreType.DMA,
        ],
    )(body)
    return k(snd_a, snd_b, src)


_NS = 16
_NPAD = 10240
_ROWS_PER_TILE = _NPAD // _NS


def _run_k3(msg_a, msg_b, dst, init_st):
    es = dst.shape[0]
    nch = es // _CH
    full_r = nch // _NS
    rem = nch - full_r * _NS
    assert full_r % 2 == 1

    def body(msg_a_hbm, msg_b_hbm, dst_hbm, init_hbm,
             out_hbm, acc, idx2, rows_v, sem0, sem1):
        ci = lax.axis_index("c")
        s = lax.axis_index("s")
        r0 = s * _ROWS_PER_TILE
        sems = [sem0, sem1]
        pltpu.sync_copy(init_hbm.at[ci, pl.ds(r0, _ROWS_PER_TILE)],
                        acc.at[pl.ds(r0, _ROWS_PER_TILE)])
        plsc.subcore_barrier()

        def scan_edges(msg_hbm):
            def fire(g, par):
                base = (s + g * _NS) * _CH
                pltpu.sync_copy(dst_hbm.at[pl.ds(base, _CH)], idx2.at[par])
                pltpu.async_copy(msg_hbm.at[pl.ds(base, _CH)], rows_v.at[par], sems[par])

            def drain_scatter(g, par):
                base = (s + g * _NS) * _CH
                pltpu.make_async_copy(msg_hbm.at[pl.ds(base, _CH)],
                                      rows_v.at[par], sems[par]).wait()
                pltpu.sync_copy(rows_v.at[par], acc.at[idx2.at[par]], add=True)

            fire(0, 0)

            def pair(i, _):
                g0 = 2 * i
                fire(g0 + 1, 1)
                drain_scatter(g0, 0)
                fire(g0 + 2, 0)
                drain_scatter(g0 + 1, 1)
                return _

            lax.fori_loop(0, (full_r - 1) // 2, pair, None)
            drain_scatter(full_r - 1, 0)

            @pl.when(s < rem)
            def _():
                base = (full_r * _NS + s) * _CH
                pltpu.sync_copy(dst_hbm.at[pl.ds(base, _CH)], idx2.at[0])
                pltpu.sync_copy(msg_hbm.at[pl.ds(base, _CH)], rows_v.at[0])
                pltpu.sync_copy(rows_v.at[0], acc.at[idx2.at[0]], add=True)

        @pl.when(ci == 0)
        def _():
            scan_edges(msg_a_hbm)

        @pl.when(ci == 1)
        def _():
            scan_edges(msg_b_hbm)

        plsc.subcore_barrier()
        pltpu.sync_copy(acc.at[pl.ds(r0, _ROWS_PER_TILE)],
                        out_hbm.at[ci, pl.ds(r0, _ROWS_PER_TILE)])

    mesh = plsc.VectorSubcoreMesh(core_axis_name="c", subcore_axis_name="s")
    k = functools.partial(
        pl.kernel,
        out_type=jax.ShapeDtypeStruct((2, _NPAD, _WA), jnp.float32),
        mesh=mesh,
        scratch_types=[
            pltpu.VMEM_SHARED((_NPAD, _WA), jnp.float32),
            pltpu.VMEM((2, _CH), jnp.int32),
            pltpu.VMEM((2, _CH, _WA), jnp.float32),
            pltpu.SemaphoreType.DMA,
            pltpu.SemaphoreType.DMA,
        ],
    )(body)
    return k(msg_a, msg_b, dst, init_st)


_NNZ_P = np.array([p for (p, i, j, k, v) in _TP_NNZ])
_NNZ_K = np.array([k for (p, i, j, k, v) in _TP_NNZ])
_NNZ_C = np.array([i * LM + j for (p, i, j, k, v) in _TP_NNZ])
_NNZ_V = np.array([v for (p, i, j, k, v) in _TP_NNZ], np.float32)


def _selfmix_matrix(pw, b0, kk):
    A = jnp.zeros((LM, 91), jnp.float32)
    A = A.at[_NNZ_K, _NNZ_C].add(_NNZ_V * pw[_NNZ_P])
    A = A.at[np.arange(LM), 81 + np.arange(LM)].add(kk[np.array(_IR_OF)])
    A = A.at[0, 90].add(b0[0])
    return A


def _prep_small(sma_path_w, sma_bias0, sma_k, cma_W, cma_b, lina_W, lina_b,
                smb_path_w, smb_bias0, smb_k, cmb_W, cmb_b, linb_W, linb_b):
    ir = jnp.asarray(_IR_OF)
    NP = len(_PATHS)
    AB = jnp.concatenate([_selfmix_matrix(sma_path_w, sma_bias0, sma_k[0]),
                          _selfmix_matrix(smb_path_w, smb_bias0, smb_k[0])], axis=0)
    WBcol = cmb_W[ir, 0, :].reshape(C * LM, 1)
    BBcol = jnp.concatenate([cmb_b, jnp.zeros(C * (LM - 1), jnp.float32)]).reshape(C * LM, 1)
    linaWr = lina_W.reshape(N_DIST, C, N_IRREPS).transpose(2, 1, 0).reshape(N_IRREPS * C, N_DIST)
    linaBr = lina_b.reshape(C, N_IRREPS).T.reshape(N_IRREPS * C, 1)
    scaleA = cma_W[:, 0, :].reshape(N_IRREPS * C, 1)
    lina2Wr = jnp.concatenate([linaWr * scaleA, linaWr[0:C]], axis=0)
    lina2Br = jnp.concatenate([linaBr * scaleA, linaBr[0:C]], axis=0)
    cmab_col = cma_b.reshape(C, 1)
    linbWr = linb_W.reshape(N_DIST, C, NP).transpose(2, 1, 0).reshape(NP * C, N_DIST)
    linbBr = linb_b.reshape(C, NP).T.reshape(NP * C, 1)
    return [AB, lina2Wr, lina2Br, cmab_col, WBcol, BBcol, linbWr, linbBr]


def kernel(nodes_rec, nodes_snd, edge_ind, Y_edge, dist_feat,
           sma_path_w, sma_bias0, sma_k, cma_W, cma_b, lina_W, lina_b,
           smb_path_w, smb_bias0, smb_k, cmb_W, cmb_b, linb_W, linb_b):
    src = edge_ind[:, 0].astype(jnp.int32)
    dst = edge_ind[:, 1].astype(jnp.int32)
    snd_flat = nodes_snd.transpose(0, 2, 1).reshape(N_NODES, C * LM)
    rec_flat = nodes_rec.transpose(0, 2, 1).reshape(N_NODES, C * LM)
    snd_a = snd_flat[:, :_WA]
    snd_b = jnp.pad(snd_flat[:, _WA:], ((0, 0), (0, _WA - _WB)))

    small = _prep_small(sma_path_w, sma_bias0, sma_k, cma_W, cma_b, lina_W, lina_b,
                        smb_path_w, smb_bias0, smb_k, cmb_W, cmb_b, linb_W, linb_b)
    rec_a = jnp.pad(rec_flat[:, :_WA], ((0, _NPAD - N_NODES), (0, 0)))
    rec_b = jnp.pad(rec_flat[:, _WA:], ((0, _NPAD - N_NODES), (0, _WA - _WB)))
    y2 = Y_edge.reshape(N_EDGES, LM)

    half = N_EDGES // 2
    state = jnp.stack([rec_a, rec_b])
    for lo in (0, half):
        sl = slice(lo, lo + half)
        nb_a, nb_b = _run_k1(snd_a, snd_b, src[sl])
        msg_a, msg_b = _run_k2(nb_a, nb_b, y2[sl].T, dist_feat[sl].T, small)
        state = _run_k3(msg_a, msg_b, dst[sl], state)
    out_flat = jnp.concatenate(
        [state[0, :N_NODES], state[1, :N_NODES, :_WB]], axis=1)
    return out_flat.reshape(N_NODES, LM, C).transpose(0, 2, 1)
